# baseline (speedup 1.0000x reference)
import functools

import jax
import jax.numpy as jnp
from jax import lax
from jax.experimental import pallas as pl
from jax.experimental.pallas import tpu as pltpu
from jax.experimental.pallas import tpu_sc as plsc

_K = 6
_NC = 2
_NS = 16
_NW = _NC * _NS
_CH = 120
_GSUB = 2
_NBUF = 3


def _knn_body(x_ref, xt_ref, wa_ref, wb_ref, b1_ref, idx_ref, a_ref, bm_ref):
    q = x_ref[...]
    xt = xt_ref[...]
    x2 = jnp.sum(xt * xt, axis=0, keepdims=True)
    q2 = jnp.sum(q * q, axis=1, keepdims=True)
    mm = jnp.dot(q, xt)
    BQ, n = mm.shape
    INF = jnp.float32(jnp.inf)
    BIG = jnp.int32(2**30)

    nfull = n // 128
    nch = nfull + (1 if n % 128 else 0)
    m1 = m2 = m3 = m4 = jnp.full((BQ, 128), INF)
    a1 = a2 = a3 = a4 = jnp.zeros((BQ, 128), jnp.int32)
    for c in range(nch):
        if c < nfull:
            sl = slice(c * 128, (c + 1) * 128)
            x = (q2 - 2.0 * mm[:, sl]) + x2[:, sl]
        else:
            rem = n - nfull * 128
            xr = (q2 - 2.0 * mm[:, nfull * 128:]) + x2[:, nfull * 128:]
            x = jnp.concatenate(
                [xr, jnp.full((BQ, 128 - rem), INF)], axis=1)
        cc = jnp.int32(c)
        c1 = x < m1
        c2 = x < m2
        c3 = x < m3
        c4 = x < m4
        m1, m2, m3, m4, a1, a2, a3, a4 = (
            jnp.where(c1, x, m1),
            jnp.where(c1, m1, jnp.where(c2, x, m2)),
            jnp.where(c2, m2, jnp.where(c3, x, m3)),
            jnp.where(c3, m3, jnp.where(c4, x, m4)),
            jnp.where(c1, cc, a1),
            jnp.where(c1, a1, jnp.where(c2, cc, a2)),
            jnp.where(c2, a2, jnp.where(c3, cc, a3)),
            jnp.where(c3, a3, jnp.where(c4, cc, a4)),
        )

    lane = lax.broadcasted_iota(jnp.int32, (BQ, 128), 1)
    col1 = a1 * 128 + lane
    col2 = a2 * 128 + lane
    col3 = a3 * 128 + lane
    col4 = a4 * 128 + lane
    pc = jnp.zeros((BQ, 128), jnp.int32)
    cols = []
    for _ in range(_K):
        gv = jnp.min(m1, axis=1, keepdims=True)
        wc = jnp.min(jnp.where(m1 == gv, col1, BIG), axis=1, keepdims=True)
        cols.append(wc)
        win = col1 == wc
        m1 = jnp.where(win, m2, m1)
        col1 = jnp.where(win, col2, col1)
        m2 = jnp.where(win, m3, m2)
        col2 = jnp.where(win, col3, col2)
        m3 = jnp.where(win, m4, m3)
        col3 = jnp.where(win, col4, col3)
        m4 = jnp.where(win, INF, m4)
        pc = pc + win.astype(jnp.int32)
    fast_idx = jnp.concatenate(cols, axis=1)

    def _slow(_):
        dd = (q2 - 2.0 * mm) + x2
        iota = lax.broadcasted_iota(jnp.int32, dd.shape, 1)
        outs = []
        for _ in range(_K):
            mv = jnp.min(dd, axis=1, keepdims=True)
            ik = jnp.min(jnp.where(dd == mv, iota, BIG),
                         axis=1, keepdims=True)
            outs.append(ik)
            dd = jnp.where(iota == ik, INF, dd)
        return jnp.concatenate(outs, axis=1)

    idx6 = lax.cond(jnp.any(pc >= 4), _slow, lambda _: fast_idx, None)
    zero = jnp.zeros((BQ, 2), jnp.int32)
    idx_ref[...] = jnp.concatenate([idx6, zero], axis=1)
    a_ref[...] = jnp.dot(q, wa_ref[...]) + b1_ref[...]
    bm_ref[...] = jnp.dot(q, wb_ref[...])


def _edge_body(a_ref, g_ref, w2_ref, b2_ref, out_ref):
    a = a_ref[...]
    BN, H = a.shape
    g = g_ref[...]
    h = jnp.maximum(g + a[None], 0.0).reshape(_K * BN, H)
    mm = jnp.dot(h, w2_ref[...]).reshape(_K, BN, H)
    out_ref[...] = jnp.max(mm, axis=0) + b2_ref[...]


def _sc_gather(bm, idx_pad):
    B = idx_pad.shape[0]
    H = bm.shape[1]
    bpw = B // _NW
    stage = _GSUB * _CH
    nst = bpw // stage
    mesh = plsc.VectorSubcoreMesh(core_axis_name="c", subcore_axis_name="s")

    @functools.partial(
        pl.kernel,
        out_type=jax.ShapeDtypeStruct((B, H), jnp.float32),
        mesh=mesh,
        scratch_types=[
            pltpu.VMEM((bpw,), jnp.int32),
        ] + [pltpu.VMEM((stage, H), jnp.float32)] * _NBUF + [
            pltpu.SemaphoreType.DMA,
            pltpu.SemaphoreType.DMA,
        ],
    )
    def run(bm_hbm, idx_hbm, out_hbm, idx_v, *rest):
        bufs, (gsem, ssem) = rest[:_NBUF], rest[_NBUF:]
        wid = lax.axis_index("s") * _NC + lax.axis_index("c")
        base = wid * bpw
        pltpu.sync_copy(idx_hbm.at[pl.ds(base, bpw)], idx_v)

        def fire(s):
            buf = bufs[s % _NBUF]
            return [
                pltpu.async_copy(
                    bm_hbm.at[idx_v.at[pl.ds(s * stage + j * _CH, _CH)]],
                    buf.at[pl.ds(j * _CH, _CH)], gsem)
                for j in range(_GSUB)
            ]

        gcp = [fire(s) for s in range(min(_NBUF, nst))]
        outcp = []
        for s in range(nst):
            for g in gcp[s]:
                g.wait()
            outcp.append(pltpu.async_copy(
                bufs[s % _NBUF],
                out_hbm.at[pl.ds(base + s * stage, stage)], ssem))
            nxt = s + _NBUF
            if nxt < nst:
                outcp[s].wait()
                gcp.append(fire(nxt))
        for cp in outcp[max(0, nst - _NBUF):]:
            cp.wait()

    return run(bm, idx_pad)


def kernel(x, edge_index, W1, b1, W2, b2):
    N, D = x.shape
    H = W2.shape[1]
    BQ = 400
    seg = 10240

    xt = x.T
    wa = W1[:D] - W1[D:]
    wb = W1[D:]
    b1r = b1.reshape(1, H)
    b2r = b2.reshape(1, H)

    idx8, A, Bm = pl.pallas_call(
        _knn_body,
        grid=(N // BQ,),
        in_specs=[
            pl.BlockSpec((BQ, D), lambda i: (i, 0)),
            pl.BlockSpec((D, N), lambda i: (0, 0)),
            pl.BlockSpec((D, H), lambda i: (0, 0)),
            pl.BlockSpec((D, H), lambda i: (0, 0)),
            pl.BlockSpec((1, H), lambda i: (0, 0)),
        ],
        out_specs=[
            pl.BlockSpec((BQ, 8), lambda i: (i, 0)),
            pl.BlockSpec((BQ, H), lambda i: (i, 0)),
            pl.BlockSpec((BQ, H), lambda i: (i, 0)),
        ],
        out_shape=[
            jax.ShapeDtypeStruct((N, 8), jnp.int32),
            jax.ShapeDtypeStruct((N, H), jnp.float32),
            jax.ShapeDtypeStruct((N, H), jnp.float32),
        ],
    )(x, xt, wa, wb, b1r)

    idx_t = idx8[:, :_K].T
    idx_pad = jnp.pad(idx_t, ((0, 0), (0, seg - N))).reshape(-1)
    G = _sc_gather(Bm, idx_pad)
    Gr = G.reshape(_K, seg, H)

    out = pl.pallas_call(
        _edge_body,
        grid=(N // BQ,),
        in_specs=[
            pl.BlockSpec((BQ, H), lambda i: (i, 0)),
            pl.BlockSpec((_K, BQ, H), lambda i: (0, i, 0)),
            pl.BlockSpec((H, H), lambda i: (0, 0)),
            pl.BlockSpec((1, H), lambda i: (0, 0)),
        ],
        out_specs=pl.BlockSpec((BQ, H), lambda i: (i, 0)),
        out_shape=jax.ShapeDtypeStruct((N, H), jnp.float32),
    )(A, Gr, W2, b2r)
    return out


# speedup vs baseline: 1.0011x; 1.0011x over previous
import functools

import jax
import jax.numpy as jnp
from jax import lax
from jax.experimental import pallas as pl
from jax.experimental.pallas import tpu as pltpu
from jax.experimental.pallas import tpu_sc as plsc

_K = 6
_NC = 2
_NS = 16
_NW = _NC * _NS
_CH = 120
_GSUB = 2
_NBUF = 3


def _knn_body(x_ref, xt_ref, wa_ref, wb_ref, b1_ref, idx_ref, a_ref, bm_ref):
    q = x_ref[...]
    xt = xt_ref[...]
    x2 = jnp.sum(xt * xt, axis=0, keepdims=True)
    q2 = jnp.sum(q * q, axis=1, keepdims=True)
    mm = jnp.dot(q, xt)
    BQ, n = mm.shape
    INF = jnp.float32(jnp.inf)
    BIG = jnp.int32(2**30)

    nfull = n // 128
    nch = nfull + (1 if n % 128 else 0)
    m1 = m2 = m3 = m4 = jnp.full((BQ, 128), INF)
    a1 = a2 = a3 = a4 = jnp.zeros((BQ, 128), jnp.int32)
    for c in range(nch):
        if c < nfull:
            sl = slice(c * 128, (c + 1) * 128)
            x = (q2 - 2.0 * mm[:, sl]) + x2[:, sl]
        else:
            rem = n - nfull * 128
            xr = (q2 - 2.0 * mm[:, nfull * 128:]) + x2[:, nfull * 128:]
            x = jnp.concatenate(
                [xr, jnp.full((BQ, 128 - rem), INF)], axis=1)
        cc = jnp.int32(c)
        c1 = x < m1
        c2 = x < m2
        c3 = x < m3
        c4 = x < m4
        m1, m2, m3, m4, a1, a2, a3, a4 = (
            jnp.where(c1, x, m1),
            jnp.where(c1, m1, jnp.where(c2, x, m2)),
            jnp.where(c2, m2, jnp.where(c3, x, m3)),
            jnp.where(c3, m3, jnp.where(c4, x, m4)),
            jnp.where(c1, cc, a1),
            jnp.where(c1, a1, jnp.where(c2, cc, a2)),
            jnp.where(c2, a2, jnp.where(c3, cc, a3)),
            jnp.where(c3, a3, jnp.where(c4, cc, a4)),
        )

    lane = lax.broadcasted_iota(jnp.int32, (BQ, 128), 1)
    col1 = a1 * 128 + lane
    col2 = a2 * 128 + lane
    col3 = a3 * 128 + lane
    col4 = a4 * 128 + lane
    pc = jnp.zeros((BQ, 128), jnp.int32)
    cols = []
    for _ in range(_K):
        gv = jnp.min(m1, axis=1, keepdims=True)
        wc = jnp.min(jnp.where(m1 == gv, col1, BIG), axis=1, keepdims=True)
        cols.append(wc)
        win = col1 == wc
        m1 = jnp.where(win, m2, m1)
        col1 = jnp.where(win, col2, col1)
        m2 = jnp.where(win, m3, m2)
        col2 = jnp.where(win, col3, col2)
        m3 = jnp.where(win, m4, m3)
        col3 = jnp.where(win, col4, col3)
        m4 = jnp.where(win, INF, m4)
        pc = pc + win.astype(jnp.int32)
    fast_idx = jnp.concatenate(cols, axis=1)

    def _slow(_):
        dd = (q2 - 2.0 * mm) + x2
        iota = lax.broadcasted_iota(jnp.int32, dd.shape, 1)
        outs = []
        for _ in range(_K):
            mv = jnp.min(dd, axis=1, keepdims=True)
            ik = jnp.min(jnp.where(dd == mv, iota, BIG),
                         axis=1, keepdims=True)
            outs.append(ik)
            dd = jnp.where(iota == ik, INF, dd)
        return jnp.concatenate(outs, axis=1)

    idx6 = lax.cond(jnp.any(pc >= 4), _slow, lambda _: fast_idx, None)
    zero = jnp.zeros((BQ, 2), jnp.int32)
    idx_ref[...] = jnp.concatenate([idx6, zero], axis=1)
    a_ref[...] = jnp.dot(q, wa_ref[...]) + b1_ref[...]
    bm_ref[...] = jnp.dot(q, wb_ref[...])


def _edge_body(a_ref, g_ref, w2_ref, b2_ref, out_ref):
    a = a_ref[...]
    acc = None
    for k in range(_K):
        h = jnp.maximum(a + g_ref[k], 0.0)
        mm = jnp.dot(h, w2_ref[...])
        acc = mm if acc is None else jnp.maximum(acc, mm)
    out_ref[...] = acc + b2_ref[...]


def _sc_gather(bm, idx_pad):
    B = idx_pad.shape[0]
    H = bm.shape[1]
    bpw = B // _NW
    stage = _GSUB * _CH
    nst = bpw // stage
    mesh = plsc.VectorSubcoreMesh(core_axis_name="c", subcore_axis_name="s")

    @functools.partial(
        pl.kernel,
        out_type=jax.ShapeDtypeStruct((B, H), jnp.float32),
        mesh=mesh,
        scratch_types=[
            pltpu.VMEM((bpw,), jnp.int32),
        ] + [pltpu.VMEM((stage, H), jnp.float32)] * _NBUF + [
            pltpu.SemaphoreType.DMA,
            pltpu.SemaphoreType.DMA,
        ],
    )
    def run(bm_hbm, idx_hbm, out_hbm, idx_v, *rest):
        bufs, (gsem, ssem) = rest[:_NBUF], rest[_NBUF:]
        wid = lax.axis_index("s") * _NC + lax.axis_index("c")
        base = wid * bpw
        pltpu.sync_copy(idx_hbm.at[pl.ds(base, bpw)], idx_v)

        def fire(s):
            buf = bufs[s % _NBUF]
            return [
                pltpu.async_copy(
                    bm_hbm.at[idx_v.at[pl.ds(s * stage + j * _CH, _CH)]],
                    buf.at[pl.ds(j * _CH, _CH)], gsem)
                for j in range(_GSUB)
            ]

        gcp = [fire(s) for s in range(min(_NBUF, nst))]
        outcp = []
        for s in range(nst):
            for g in gcp[s]:
                g.wait()
            outcp.append(pltpu.async_copy(
                bufs[s % _NBUF],
                out_hbm.at[pl.ds(base + s * stage, stage)], ssem))
            nxt = s + _NBUF
            if nxt < nst:
                outcp[s].wait()
                gcp.append(fire(nxt))
        for cp in outcp[max(0, nst - _NBUF):]:
            cp.wait()

    return run(bm, idx_pad)


def kernel(x, edge_index, W1, b1, W2, b2):
    N, D = x.shape
    H = W2.shape[1]
    BQ = 400
    seg = 10240

    xt = x.T
    wa = W1[:D] - W1[D:]
    wb = W1[D:]
    b1r = b1.reshape(1, H)
    b2r = b2.reshape(1, H)

    idx8, A, Bm = pl.pallas_call(
        _knn_body,
        grid=(N // BQ,),
        in_specs=[
            pl.BlockSpec((BQ, D), lambda i: (i, 0)),
            pl.BlockSpec((D, N), lambda i: (0, 0)),
            pl.BlockSpec((D, H), lambda i: (0, 0)),
            pl.BlockSpec((D, H), lambda i: (0, 0)),
            pl.BlockSpec((1, H), lambda i: (0, 0)),
        ],
        out_specs=[
            pl.BlockSpec((BQ, 8), lambda i: (i, 0)),
            pl.BlockSpec((BQ, H), lambda i: (i, 0)),
            pl.BlockSpec((BQ, H), lambda i: (i, 0)),
        ],
        out_shape=[
            jax.ShapeDtypeStruct((N, 8), jnp.int32),
            jax.ShapeDtypeStruct((N, H), jnp.float32),
            jax.ShapeDtypeStruct((N, H), jnp.float32),
        ],
    )(x, xt, wa, wb, b1r)

    idx_t = idx8[:, :_K].T
    idx_pad = jnp.pad(idx_t, ((0, 0), (0, seg - N))).reshape(-1)
    G = _sc_gather(Bm, idx_pad)
    Gr = G.reshape(_K, seg, H)

    out = pl.pallas_call(
        _edge_body,
        grid=(N // BQ,),
        in_specs=[
            pl.BlockSpec((BQ, H), lambda i: (i, 0)),
            pl.BlockSpec((_K, BQ, H), lambda i: (0, i, 0)),
            pl.BlockSpec((H, H), lambda i: (0, 0)),
            pl.BlockSpec((1, H), lambda i: (0, 0)),
        ],
        out_specs=pl.BlockSpec((BQ, H), lambda i: (i, 0)),
        out_shape=jax.ShapeDtypeStruct((N, H), jnp.float32),
    )(A, Gr, W2, b2r)
    return out


# speedup vs baseline: 1.0230x; 1.0219x over previous
import functools

import jax
import jax.numpy as jnp
from jax import lax
from jax.experimental import pallas as pl
from jax.experimental.pallas import tpu as pltpu
from jax.experimental.pallas import tpu_sc as plsc

_K = 6
_NC = 2
_NS = 16
_NW = _NC * _NS
_CH = 120
_GSUB = 2
_NBUF = 3


def _knn_body(x_ref, xt_ref, wa_ref, wb_ref, b1_ref, idx_ref, a_ref, bm_ref):
    q = x_ref[...]
    xt = xt_ref[...]
    x2 = jnp.sum(xt * xt, axis=0, keepdims=True)
    q2 = jnp.sum(q * q, axis=1, keepdims=True)
    mm = jnp.dot(q, xt)
    BQ, n = mm.shape
    INF = jnp.float32(jnp.inf)
    BIG = jnp.int32(2**30)

    nfull = n // 128
    nch = nfull + (1 if n % 128 else 0)
    m1 = m2 = m3 = m4 = jnp.full((BQ, 128), INF)
    a1 = a2 = a3 = a4 = jnp.zeros((BQ, 128), jnp.int32)
    for c in range(nch):
        if c < nfull:
            sl = slice(c * 128, (c + 1) * 128)
            x = (q2 - 2.0 * mm[:, sl]) + x2[:, sl]
        else:
            rem = n - nfull * 128
            xr = (q2 - 2.0 * mm[:, nfull * 128:]) + x2[:, nfull * 128:]
            x = jnp.concatenate(
                [xr, jnp.full((BQ, 128 - rem), INF)], axis=1)
        cc = jnp.int32(c)
        c1 = x < m1
        c2 = x < m2
        c3 = x < m3
        c4 = x < m4
        m1, m2, m3, m4, a1, a2, a3, a4 = (
            jnp.where(c1, x, m1),
            jnp.where(c1, m1, jnp.where(c2, x, m2)),
            jnp.where(c2, m2, jnp.where(c3, x, m3)),
            jnp.where(c3, m3, jnp.where(c4, x, m4)),
            jnp.where(c1, cc, a1),
            jnp.where(c1, a1, jnp.where(c2, cc, a2)),
            jnp.where(c2, a2, jnp.where(c3, cc, a3)),
            jnp.where(c3, a3, jnp.where(c4, cc, a4)),
        )

    lane = lax.broadcasted_iota(jnp.int32, (BQ, 128), 1)
    col1 = a1 * 128 + lane
    col2 = a2 * 128 + lane
    col3 = a3 * 128 + lane
    col4 = a4 * 128 + lane
    pc = jnp.zeros((BQ, 128), jnp.int32)
    cols = []
    for _ in range(_K):
        gv = jnp.min(m1, axis=1, keepdims=True)
        wc = jnp.min(jnp.where(m1 == gv, col1, BIG), axis=1, keepdims=True)
        cols.append(wc)
        win = col1 == wc
        m1 = jnp.where(win, m2, m1)
        col1 = jnp.where(win, col2, col1)
        m2 = jnp.where(win, m3, m2)
        col2 = jnp.where(win, col3, col2)
        m3 = jnp.where(win, m4, m3)
        col3 = jnp.where(win, col4, col3)
        m4 = jnp.where(win, INF, m4)
        pc = pc + win.astype(jnp.int32)
    fast_idx = jnp.concatenate(cols, axis=1)

    def _slow(_):
        dd = (q2 - 2.0 * mm) + x2
        iota = lax.broadcasted_iota(jnp.int32, dd.shape, 1)
        outs = []
        for _ in range(_K):
            mv = jnp.min(dd, axis=1, keepdims=True)
            ik = jnp.min(jnp.where(dd == mv, iota, BIG),
                         axis=1, keepdims=True)
            outs.append(ik)
            dd = jnp.where(iota == ik, INF, dd)
        return jnp.concatenate(outs, axis=1)

    idx6 = lax.cond(jnp.any(pc >= 4), _slow, lambda _: fast_idx, None)
    zero = jnp.zeros((BQ, 2), jnp.int32)
    idx_ref[...] = jnp.concatenate([idx6, zero], axis=1)
    a_ref[...] = jnp.dot(q, wa_ref[...]) + b1_ref[...]
    bm_ref[...] = jnp.dot(q, wb_ref[...])


def _edge_body(a_ref, g_ref, w2_ref, b2_ref, out_ref):
    a = a_ref[...]
    acc = None
    for k in range(_K):
        h = jnp.maximum(a + g_ref[k], 0.0)
        mm = jnp.dot(h, w2_ref[...])
        acc = mm if acc is None else jnp.maximum(acc, mm)
    out_ref[...] = acc + b2_ref[...]


def _sc_gather(bm, idx_pad):
    B = idx_pad.shape[0]
    H = bm.shape[1]
    bpw = B // _NW
    stage = _GSUB * _CH
    nst = bpw // stage
    mesh = plsc.VectorSubcoreMesh(core_axis_name="c", subcore_axis_name="s")

    @functools.partial(
        pl.kernel,
        out_type=jax.ShapeDtypeStruct((B, H), jnp.float32),
        mesh=mesh,
        scratch_types=[
            pltpu.VMEM((bpw,), jnp.int32),
        ] + [pltpu.VMEM((stage, H), jnp.float32)] * _NBUF + [
            pltpu.SemaphoreType.DMA,
            pltpu.SemaphoreType.DMA,
        ],
    )
    def run(bm_hbm, idx_hbm, out_hbm, idx_v, *rest):
        bufs, (gsem, ssem) = rest[:_NBUF], rest[_NBUF:]
        wid = lax.axis_index("s") * _NC + lax.axis_index("c")
        base = wid * bpw
        pltpu.sync_copy(idx_hbm.at[pl.ds(base, bpw)], idx_v)

        def fire(s):
            buf = bufs[s % _NBUF]
            return [
                pltpu.async_copy(
                    bm_hbm.at[idx_v.at[pl.ds(s * stage + j * _CH, _CH)]],
                    buf.at[pl.ds(j * _CH, _CH)], gsem)
                for j in range(_GSUB)
            ]

        gcp = [fire(s) for s in range(min(_NBUF, nst))]
        outcp = []
        for s in range(nst):
            for g in gcp[s]:
                g.wait()
            outcp.append(pltpu.async_copy(
                bufs[s % _NBUF],
                out_hbm.at[pl.ds(base + s * stage, stage)], ssem))
            nxt = s + _NBUF
            if nxt < nst:
                outcp[s].wait()
                gcp.append(fire(nxt))
        for cp in outcp[max(0, nst - _NBUF):]:
            cp.wait()

    return run(bm, idx_pad)


def kernel(x, edge_index, W1, b1, W2, b2):
    N, D = x.shape
    H = W2.shape[1]
    BQ = 400
    seg = 10240

    xt = x.T
    wa = W1[:D] - W1[D:]
    wb = W1[D:]
    b1r = b1.reshape(1, H)
    b2r = b2.reshape(1, H)

    idx8, A, Bm = pl.pallas_call(
        _knn_body,
        grid=(N // BQ,),
        in_specs=[
            pl.BlockSpec((BQ, D), lambda i: (i, 0)),
            pl.BlockSpec((D, N), lambda i: (0, 0)),
            pl.BlockSpec((D, H), lambda i: (0, 0)),
            pl.BlockSpec((D, H), lambda i: (0, 0)),
            pl.BlockSpec((1, H), lambda i: (0, 0)),
        ],
        out_specs=[
            pl.BlockSpec((BQ, 8), lambda i: (i, 0)),
            pl.BlockSpec((BQ, H), lambda i: (i, 0)),
            pl.BlockSpec((BQ, H), lambda i: (i, 0)),
        ],
        out_shape=[
            jax.ShapeDtypeStruct((N, 8), jnp.int32),
            jax.ShapeDtypeStruct((N, H), jnp.float32),
            jax.ShapeDtypeStruct((N, H), jnp.float32),
        ],
    )(x, xt, wa, wb, b1r)

    idx_t = idx8[:, :_K].T
    idx_pad = jnp.pad(idx_t, ((0, 0), (0, seg - N))).reshape(-1)
    G = _sc_gather(Bm, idx_pad)
    Gr = G.reshape(_K, seg, H)

    BN = 2000
    out = pl.pallas_call(
        _edge_body,
        grid=(N // BN,),
        in_specs=[
            pl.BlockSpec((BN, H), lambda i: (i, 0)),
            pl.BlockSpec((_K, BN, H), lambda i: (0, i, 0)),
            pl.BlockSpec((H, H), lambda i: (0, 0)),
            pl.BlockSpec((1, H), lambda i: (0, 0)),
        ],
        out_specs=pl.BlockSpec((BN, H), lambda i: (i, 0)),
        out_shape=jax.ShapeDtypeStruct((N, H), jnp.float32),
    )(A, Gr, W2, b2r)
    return out
